# baseline (speedup 1.0000x reference)
import functools

import jax
import jax.numpy as jnp
from jax import lax
from jax.experimental import pallas as pl
from jax.experimental.pallas import tpu as pltpu
from jax.experimental.pallas import tpu_sc as plsc

N_NODES = 10000
N_EDGES = 320000
D = 128

NC = 2
NS = 16
L = 16
NW = NC * NS

CHUNK = 128
CHUNKS_PW = 80
PAD_EDGES = NW * CHUNKS_PW * CHUNK

HCHUNK = 128
N_HCHUNKS = N_NODES // HCHUNK
HROUNDS = (N_HCHUNKS + NS - 1) // NS
HTAIL = N_NODES - N_HCHUNKS * HCHUNK


def _sc_aggregate(x, rows, cols, vals):
    mesh = plsc.VectorSubcoreMesh(
        core_axis_name="c", subcore_axis_name="s",
        num_cores=NC, num_subcores=NS)

    @functools.partial(
        pl.kernel,
        out_type=jax.ShapeDtypeStruct((NC, N_NODES, D), jnp.float32),
        mesh=mesh,
        scratch_types=[
            pltpu.VMEM_SHARED((N_NODES, D), jnp.float32),
            pltpu.VMEM((CHUNK, D), jnp.float32),
            pltpu.VMEM((CHUNK, D), jnp.float32),
            pltpu.VMEM((CHUNK,), jnp.int32),
            pltpu.VMEM((CHUNK,), jnp.int32),
            pltpu.VMEM((CHUNK,), jnp.int32),
            pltpu.VMEM((CHUNK,), jnp.int32),
            pltpu.VMEM((CHUNK,), jnp.float32),
            pltpu.VMEM((CHUNK,), jnp.float32),
            pltpu.SemaphoreType.DMA,
        ],
    )
    def agg(x_hbm, rows_hbm, cols_hbm, vals_hbm, out_hbm,
            h_sh, gbuf0, gbuf1, colb0, colb1, rowb0, rowb1, valb0, valb1,
            sem):
        c = lax.axis_index("c")
        s = lax.axis_index("s")
        wid = c * NS + s
        gbuf = (gbuf0, gbuf1)
        colb = (colb0, colb1)
        rowb = (rowb0, rowb1)
        valb = (valb0, valb1)
        ebase = wid * CHUNKS_PW * CHUNK

        def zero_row(r, _):
            for j in range(D // L):
                gbuf0[r, pl.ds(j * L, L)] = jnp.zeros((L,), jnp.float32)
            return 0
        lax.fori_loop(0, HCHUNK, zero_row, 0)
        for k in range(HROUNDS):
            cid = s + NS * k

            @pl.when(cid < N_HCHUNKS)
            def _():
                pltpu.sync_copy(gbuf0, h_sh.at[pl.ds(cid * HCHUNK, HCHUNK)])

        @pl.when(s == NS - 1)
        def _():
            pltpu.sync_copy(gbuf0.at[pl.ds(0, HTAIL)],
                            h_sh.at[pl.ds(N_HCHUNKS * HCHUNK, HTAIL)])
        plsc.subcore_barrier()

        def scale_rows(gb, vb):
            def body(g, _):
                v16 = vb[pl.ds(g * L, L)]
                for i in range(L):
                    r = g * L + i
                    vs = v16[i]
                    for j in range(D // L):
                        gb[r, pl.ds(j * L, L)] = gb[r, pl.ds(j * L, L)] * vs
                return 0
            lax.fori_loop(0, CHUNK // L, body, 0)

        def fetch(k, b):
            base = ebase + k * CHUNK
            pltpu.sync_copy(cols_hbm.at[pl.ds(base, CHUNK)], colb[b])
            pltpu.sync_copy(rows_hbm.at[pl.ds(base, CHUNK)], rowb[b])
            pltpu.sync_copy(vals_hbm.at[pl.ds(base, CHUNK)], valb[b])
            pltpu.async_copy(x_hbm.at[colb[b]], gbuf[b], sem)

        def finish(k, b, prefetch):
            pltpu.make_async_copy(
                x_hbm.at[colb[b]], gbuf[b], sem).wait()
            if prefetch:
                fetch(k + 1, 1 - b)
            scale_rows(gbuf[b], valb[b])
            pltpu.sync_copy(gbuf[b], h_sh.at[rowb[b]], add=True)

        fetch(0, 0)

        def round2(o, _):
            for b in range(2):
                finish(o * 2 + b, b, prefetch=True)
            return 0
        lax.fori_loop(0, CHUNKS_PW // 2 - 1, round2, 0)
        finish(CHUNKS_PW - 2, 0, prefetch=True)
        finish(CHUNKS_PW - 1, 1, prefetch=False)

        plsc.subcore_barrier()

        for k in range(HROUNDS):
            cid = s + NS * k

            @pl.when(cid < N_HCHUNKS)
            def _():
                pltpu.sync_copy(h_sh.at[pl.ds(cid * HCHUNK, HCHUNK)],
                                out_hbm.at[c, pl.ds(cid * HCHUNK, HCHUNK)])

        @pl.when(s == NS - 1)
        def _():
            pltpu.sync_copy(h_sh.at[pl.ds(N_HCHUNKS * HCHUNK, HTAIL)],
                            out_hbm.at[c, pl.ds(N_HCHUNKS * HCHUNK, HTAIL)])

    return agg(x, rows, cols, vals)


def _tc_matmul_relu(h_partial, W):
    BLOCK = 1000

    def mm(h_ref, w_ref, o_ref):
        hp = h_ref[...]
        y = hp[0] + hp[1]
        o_ref[...] = jnp.maximum(
            jnp.dot(y, w_ref[...], preferred_element_type=jnp.float32), 0.0)

    return pl.pallas_call(
        mm,
        grid=(N_NODES // BLOCK,),
        in_specs=[
            pl.BlockSpec((NC, BLOCK, D), lambda i: (0, i, 0)),
            pl.BlockSpec((D, D), lambda i: (0, 0)),
        ],
        out_specs=pl.BlockSpec((BLOCK, D), lambda i: (i, 0)),
        out_shape=jax.ShapeDtypeStruct((N_NODES, D), jnp.float32),
    )(h_partial, W)


def _pad_edges(adj_indices, adj_values):
    rows = adj_indices[0]
    cols = adj_indices[1]
    pad = PAD_EDGES - N_EDGES
    zpad = jnp.zeros((pad,), jnp.int32)
    rows_p = jnp.concatenate([rows, zpad])
    cols_p = jnp.concatenate([cols, zpad])
    vals_p = jnp.concatenate([adj_values, jnp.zeros((pad,), jnp.float32)])
    return rows_p, cols_p, vals_p


def kernel(input, adj_indices, adj_values, W):
    rows_p, cols_p, vals_p = _pad_edges(adj_indices, adj_values)
    h_partial = _sc_aggregate(input, rows_p, cols_p, vals_p)
    return _tc_matmul_relu(h_partial, W)


# speedup vs baseline: 2.4789x; 2.4789x over previous
import functools

import jax
import jax.numpy as jnp
from jax import lax
from jax.experimental import pallas as pl
from jax.experimental.pallas import tpu as pltpu
from jax.experimental.pallas import tpu_sc as plsc

N_NODES = 10000
N_EDGES = 320000
D = 128

NC = 2
NS = 16
L = 16
NW = NC * NS

CHUNK = 128
CHUNKS_PW = 80
PAD_EDGES = NW * CHUNKS_PW * CHUNK

HCHUNK = 128
N_HCHUNKS = N_NODES // HCHUNK
HROUNDS = (N_HCHUNKS + NS - 1) // NS
HTAIL = N_NODES - N_HCHUNKS * HCHUNK


def _sc_aggregate(x, rows, cols, vals):
    mesh = plsc.VectorSubcoreMesh(
        core_axis_name="c", subcore_axis_name="s",
        num_cores=NC, num_subcores=NS)

    @functools.partial(
        pl.kernel,
        out_type=jax.ShapeDtypeStruct((NC, N_NODES, D), jnp.float32),
        mesh=mesh,
        scratch_types=[
            pltpu.VMEM_SHARED((N_NODES, D), jnp.float32),
            pltpu.VMEM((CHUNK, D), jnp.float32),
            pltpu.VMEM((CHUNK, D), jnp.float32),
            pltpu.VMEM((CHUNK,), jnp.int32),
            pltpu.VMEM((CHUNK,), jnp.int32),
            pltpu.VMEM((CHUNK,), jnp.int32),
            pltpu.VMEM((CHUNK,), jnp.int32),
            pltpu.VMEM((CHUNK,), jnp.float32),
            pltpu.VMEM((CHUNK,), jnp.float32),
            pltpu.SemaphoreType.DMA,
        ],
    )
    def agg(x_hbm, rows_hbm, cols_hbm, vals_hbm, out_hbm,
            h_sh, gbuf0, gbuf1, colb0, colb1, rowb0, rowb1, valb0, valb1,
            sem):
        c = lax.axis_index("c")
        s = lax.axis_index("s")
        wid = c * NS + s
        gbuf = (gbuf0, gbuf1)
        colb = (colb0, colb1)
        rowb = (rowb0, rowb1)
        valb = (valb0, valb1)
        ebase = wid * CHUNKS_PW * CHUNK

        def zero_row(r, _):
            for j in range(D // L):
                gbuf0[r, pl.ds(j * L, L)] = jnp.zeros((L,), jnp.float32)
            return 0
        lax.fori_loop(0, HCHUNK, zero_row, 0)
        for k in range(HROUNDS):
            cid = s + NS * k

            @pl.when(cid < N_HCHUNKS)
            def _():
                pltpu.sync_copy(gbuf0, h_sh.at[pl.ds(cid * HCHUNK, HCHUNK)])

        @pl.when(s == NS - 1)
        def _():
            pltpu.sync_copy(gbuf0.at[pl.ds(0, HTAIL)],
                            h_sh.at[pl.ds(N_HCHUNKS * HCHUNK, HTAIL)])
        plsc.subcore_barrier()

        def scale_rows(gb, vb):
            def body(g, _):
                v16 = vb[pl.ds(g * L, L)]
                for i in range(L):
                    r = g * L + i
                    vs = v16[i]
                    for j in range(D // L):
                        gb[r, pl.ds(j * L, L)] = gb[r, pl.ds(j * L, L)] * vs
                return 0
            lax.fori_loop(0, CHUNK // L, body, 0)

        def fetch(k, b):
            base = ebase + k * CHUNK
            pltpu.sync_copy(cols_hbm.at[pl.ds(base, CHUNK)], colb[b])
            pltpu.sync_copy(rows_hbm.at[pl.ds(base, CHUNK)], rowb[b])
            pltpu.sync_copy(vals_hbm.at[pl.ds(base, CHUNK)], valb[b])
            pltpu.async_copy(x_hbm.at[colb[b]], gbuf[b], sem)

        def finish(k, b, prefetch):
            pltpu.make_async_copy(
                x_hbm.at[colb[b]], gbuf[b], sem).wait()
            if prefetch:
                fetch(k + 1, 1 - b)
            scale_rows(gbuf[b], valb[b])
            pltpu.sync_copy(gbuf[b], h_sh.at[rowb[b]], add=True)

        fetch(0, 0)

        def round2(o, _):
            for b in range(2):
                finish(o * 2 + b, b, prefetch=True)
            return 0
        lax.fori_loop(0, CHUNKS_PW // 2 - 1, round2, 0)
        finish(CHUNKS_PW - 2, 0, prefetch=True)
        finish(CHUNKS_PW - 1, 1, prefetch=False)

        plsc.subcore_barrier()

        for k in range(HROUNDS):
            cid = s + NS * k

            @pl.when(cid < N_HCHUNKS)
            def _():
                pltpu.sync_copy(h_sh.at[pl.ds(cid * HCHUNK, HCHUNK)],
                                out_hbm.at[c, pl.ds(cid * HCHUNK, HCHUNK)])

        @pl.when(s == NS - 1)
        def _():
            pltpu.sync_copy(h_sh.at[pl.ds(N_HCHUNKS * HCHUNK, HTAIL)],
                            out_hbm.at[c, pl.ds(N_HCHUNKS * HCHUNK, HTAIL)])

    return agg(x, rows, cols, vals)


def _tc_matmul_relu(h_partial, W):
    BLOCK = 1000

    def mm(h_ref, w_ref, o_ref):
        hp = h_ref[...]
        y = hp[0] + hp[1]
        o_ref[...] = jnp.maximum(
            jnp.dot(y, w_ref[...], preferred_element_type=jnp.float32), 0.0)

    return pl.pallas_call(
        mm,
        grid=(N_NODES // BLOCK,),
        in_specs=[
            pl.BlockSpec((NC, BLOCK, D), lambda i: (0, i, 0)),
            pl.BlockSpec((D, D), lambda i: (0, 0)),
        ],
        out_specs=pl.BlockSpec((BLOCK, D), lambda i: (i, 0)),
        out_shape=jax.ShapeDtypeStruct((N_NODES, D), jnp.float32),
    )(h_partial, W)


def _pad_edges(adj_indices, adj_values):
    rows = adj_indices[0]
    cols = adj_indices[1]
    pad = PAD_EDGES - N_EDGES
    spread = jnp.arange(pad, dtype=jnp.int32) % N_NODES
    rows_p = jnp.concatenate([rows, spread])
    cols_p = jnp.concatenate([cols, spread])
    vals_p = jnp.concatenate([adj_values, jnp.zeros((pad,), jnp.float32)])
    return rows_p, cols_p, vals_p


def kernel(input, adj_indices, adj_values, W):
    rows_p, cols_p, vals_p = _pad_edges(adj_indices, adj_values)
    h_partial = _sc_aggregate(input, rows_p, cols_p, vals_p)
    return _tc_matmul_relu(h_partial, W)


# speedup vs baseline: 2.5300x; 1.0206x over previous
import functools

import jax
import jax.numpy as jnp
from jax import lax
from jax.experimental import pallas as pl
from jax.experimental.pallas import tpu as pltpu
from jax.experimental.pallas import tpu_sc as plsc

N_NODES = 10000
N_EDGES = 320000
D = 128

NC = 2
NS = 16
L = 16
NW = NC * NS

CHUNK = 128
CHUNKS_PW = 80
PAD_EDGES = NW * CHUNKS_PW * CHUNK
NBUF = 3

HCHUNK = 128
N_HCHUNKS = N_NODES // HCHUNK
HROUNDS = (N_HCHUNKS + NS - 1) // NS
HTAIL = N_NODES - N_HCHUNKS * HCHUNK


def _sc_aggregate(x, rows2d, cols2d, vals2d):
    mesh = plsc.VectorSubcoreMesh(
        core_axis_name="c", subcore_axis_name="s",
        num_cores=NC, num_subcores=NS)

    @functools.partial(
        pl.kernel,
        out_type=jax.ShapeDtypeStruct((NC, N_NODES, D), jnp.float32),
        mesh=mesh,
        scratch_types=[
            pltpu.VMEM_SHARED((N_NODES, D), jnp.float32),
            pltpu.VMEM((CHUNK, D), jnp.float32),
            pltpu.VMEM((CHUNK, D), jnp.float32),
            pltpu.VMEM((CHUNK, D), jnp.float32),
            pltpu.VMEM((CHUNK,), jnp.int32),
            pltpu.VMEM((CHUNK,), jnp.int32),
            pltpu.VMEM((CHUNK,), jnp.int32),
            pltpu.VMEM((CHUNK,), jnp.int32),
            pltpu.VMEM((CHUNK,), jnp.int32),
            pltpu.VMEM((CHUNK,), jnp.int32),
            pltpu.VMEM((CHUNK,), jnp.float32),
            pltpu.VMEM((CHUNK,), jnp.float32),
            pltpu.VMEM((CHUNK,), jnp.float32),
            pltpu.SemaphoreType.DMA,
        ],
    )
    def agg(x_hbm, rows_hbm, cols_hbm, vals_hbm, out_hbm,
            h_sh, gbuf0, gbuf1, gbuf2, colb0, colb1, colb2,
            rowb0, rowb1, rowb2, valb0, valb1, valb2, sem):
        c = lax.axis_index("c")
        s = lax.axis_index("s")
        wid = c * NS + s
        gbuf = (gbuf0, gbuf1, gbuf2)
        colb = (colb0, colb1, colb2)
        rowb = (rowb0, rowb1, rowb2)
        valb = (valb0, valb1, valb2)

        def zero_row(r, _):
            for j in range(D // L):
                gbuf0[r, pl.ds(j * L, L)] = jnp.zeros((L,), jnp.float32)
            return 0
        lax.fori_loop(0, HCHUNK, zero_row, 0)
        for k in range(HROUNDS):
            cid = s + NS * k

            @pl.when(cid < N_HCHUNKS)
            def _():
                pltpu.sync_copy(gbuf0, h_sh.at[pl.ds(cid * HCHUNK, HCHUNK)])

        @pl.when(s == NS - 1)
        def _():
            pltpu.sync_copy(gbuf0.at[pl.ds(0, HTAIL)],
                            h_sh.at[pl.ds(N_HCHUNKS * HCHUNK, HTAIL)])
        plsc.subcore_barrier()

        def scale_rows(gb, vb):
            def body(g, _):
                v16 = vb[pl.ds(g * L, L)]
                for i in range(L):
                    r = g * L + i
                    vs = v16[i]
                    for j in range(D // L):
                        gb[r, pl.ds(j * L, L)] = gb[r, pl.ds(j * L, L)] * vs
                return 0
            lax.fori_loop(0, CHUNK // L, body, 0)

        def fetch(k, b):
            base = (wid * CHUNKS_PW + k) * CHUNK
            pltpu.sync_copy(cols_hbm.at[pl.ds(base, CHUNK)], colb[b])
            pltpu.sync_copy(rows_hbm.at[pl.ds(base, CHUNK)], rowb[b])
            pltpu.sync_copy(vals_hbm.at[pl.ds(base, CHUNK)], valb[b])
            pltpu.async_copy(x_hbm.at[colb[b]], gbuf[b], sem)

        for b in range(NBUF - 1):
            fetch(b, b)

        def finish(k, b, prefetch):
            pltpu.make_async_copy(
                x_hbm.at[colb[b]], gbuf[b], sem).wait()
            if prefetch:
                fetch(k + NBUF - 1, (b + NBUF - 1) % NBUF)
            scale_rows(gbuf[b], valb[b])
            pltpu.sync_copy(gbuf[b], h_sh.at[rowb[b]], add=True)

        def round3(o, _):
            for b in range(NBUF):
                finish(o * NBUF + b, b, prefetch=True)
            return 0
        lax.fori_loop(0, (CHUNKS_PW - NBUF + 1) // NBUF, round3, 0)
        finish(CHUNKS_PW - 2, (CHUNKS_PW - 2) % NBUF, prefetch=False)
        finish(CHUNKS_PW - 1, (CHUNKS_PW - 1) % NBUF, prefetch=False)

        plsc.subcore_barrier()

        for k in range(HROUNDS):
            cid = s + NS * k

            @pl.when(cid < N_HCHUNKS)
            def _():
                pltpu.sync_copy(h_sh.at[pl.ds(cid * HCHUNK, HCHUNK)],
                                out_hbm.at[c, pl.ds(cid * HCHUNK, HCHUNK)])

        @pl.when(s == NS - 1)
        def _():
            pltpu.sync_copy(h_sh.at[pl.ds(N_HCHUNKS * HCHUNK, HTAIL)],
                            out_hbm.at[c, pl.ds(N_HCHUNKS * HCHUNK, HTAIL)])

    return agg(x, rows2d, cols2d, vals2d)


def _tc_matmul_relu(h_partial, W):
    BLOCK = 1000

    def mm(h_ref, w_ref, o_ref):
        hp = h_ref[...]
        y = hp[0] + hp[1]
        o_ref[...] = jnp.maximum(
            jnp.dot(y, w_ref[...], preferred_element_type=jnp.float32), 0.0)

    return pl.pallas_call(
        mm,
        grid=(N_NODES // BLOCK,),
        in_specs=[
            pl.BlockSpec((NC, BLOCK, D), lambda i: (0, i, 0)),
            pl.BlockSpec((D, D), lambda i: (0, 0)),
        ],
        out_specs=pl.BlockSpec((BLOCK, D), lambda i: (i, 0)),
        out_shape=jax.ShapeDtypeStruct((N_NODES, D), jnp.float32),
    )(h_partial, W)


def _pad_edges(adj_indices, adj_values):
    rows = adj_indices[0]
    cols = adj_indices[1]
    pad = PAD_EDGES - N_EDGES
    spread = jnp.arange(pad, dtype=jnp.int32) % N_NODES
    rows_p = jnp.concatenate([rows, spread])
    cols_p = jnp.concatenate([cols, spread])
    vals_p = jnp.concatenate([adj_values, jnp.zeros((pad,), jnp.float32)])
    return rows_p, cols_p, vals_p


def kernel(input, adj_indices, adj_values, W):
    rows_p, cols_p, vals_p = _pad_edges(adj_indices, adj_values)
    h_partial = _sc_aggregate(input, rows_p, cols_p, vals_p)
    return _tc_matmul_relu(h_partial, W)


# speedup vs baseline: 3.3121x; 1.3091x over previous
import functools

import jax
import jax.numpy as jnp
from jax import lax
from jax.experimental import pallas as pl
from jax.experimental.pallas import tpu as pltpu
from jax.experimental.pallas import tpu_sc as plsc

N_NODES = 10000
N_EDGES = 320000
D = 128

NC = 2
NS = 16
L = 16
NW = NC * NS

CHUNK = 128
CHUNKS_PW = 80
PAD_EDGES = NW * CHUNKS_PW * CHUNK
NBUF = 3
MW = 3 * CHUNK

HCHUNK = 128
N_HCHUNKS = N_NODES // HCHUNK
HROUNDS = (N_HCHUNKS + NS - 1) // NS
HTAIL = N_NODES - N_HCHUNKS * HCHUNK


def _sc_aggregate(x, meta):
    mesh = plsc.VectorSubcoreMesh(
        core_axis_name="c", subcore_axis_name="s",
        num_cores=NC, num_subcores=NS)

    @functools.partial(
        pl.kernel,
        out_type=jax.ShapeDtypeStruct((NC, N_NODES, D), jnp.float32),
        mesh=mesh,
        scratch_types=[
            pltpu.VMEM_SHARED((N_NODES, D), jnp.float32),
            pltpu.VMEM((CHUNK, D), jnp.float32),
            pltpu.VMEM((CHUNK, D), jnp.float32),
            pltpu.VMEM((CHUNK, D), jnp.float32),
            pltpu.VMEM((MW,), jnp.int32),
            pltpu.VMEM((MW,), jnp.int32),
            pltpu.VMEM((MW,), jnp.int32),
            pltpu.VMEM((CHUNK,), jnp.int32),
            pltpu.VMEM((CHUNK,), jnp.int32),
            pltpu.VMEM((CHUNK,), jnp.int32),
            pltpu.SemaphoreType.DMA,
            pltpu.SemaphoreType.DMA,
        ],
    )
    def agg(x_hbm, meta_hbm, out_hbm,
            h_sh, gbuf0, gbuf1, gbuf2, metab0, metab1, metab2,
            ridx0, ridx1, ridx2, sem_g, sem_m):
        c = lax.axis_index("c")
        s = lax.axis_index("s")
        wid = c * NS + s
        gbuf = (gbuf0, gbuf1, gbuf2)
        metab = (metab0, metab1, metab2)
        ridx = (ridx0, ridx1, ridx2)

        def zero_row(r, _):
            for j in range(D // L):
                gbuf0[r, pl.ds(j * L, L)] = jnp.zeros((L,), jnp.float32)
            return 0
        lax.fori_loop(0, HCHUNK, zero_row, 0)
        for k in range(HROUNDS):
            cid = s + NS * k

            @pl.when(cid < N_HCHUNKS)
            def _():
                pltpu.sync_copy(gbuf0, h_sh.at[pl.ds(cid * HCHUNK, HCHUNK)])

        @pl.when(s == NS - 1)
        def _():
            pltpu.sync_copy(gbuf0.at[pl.ds(0, HTAIL)],
                            h_sh.at[pl.ds(N_HCHUNKS * HCHUNK, HTAIL)])
        plsc.subcore_barrier()

        def scale_rows(gb, mb):
            def body(g, _):
                v16 = lax.bitcast_convert_type(
                    mb[pl.ds(2 * CHUNK + g * L, L)], jnp.float32)
                for i in range(L):
                    r = g * L + i
                    vs = v16[i]
                    for j in range(D // L):
                        gb[r, pl.ds(j * L, L)] = gb[r, pl.ds(j * L, L)] * vs
                return 0
            lax.fori_loop(0, CHUNK // L, body, 0)

        def fetch_meta(k, b):
            base = (wid * CHUNKS_PW + k) * MW
            pltpu.async_copy(meta_hbm.at[pl.ds(base, MW)], metab[b], sem_m)

        def launch(k, b):
            pltpu.make_async_copy(
                meta_hbm.at[pl.ds(0, MW)], metab[b], sem_m).wait()
            for g in range(CHUNK // L):
                ridx[b][pl.ds(g * L, L)] = metab[b][
                    pl.ds(CHUNK + g * L, L)]
            pltpu.async_copy(
                x_hbm.at[metab[b].at[pl.ds(0, CHUNK)]], gbuf[b], sem_g)

        fetch_meta(0, 0)
        fetch_meta(1, 1)
        launch(0, 0)
        launch(1, 1)
        fetch_meta(2, 2)

        def finish(k, b, launch_ahead, fetch_ahead):
            pltpu.make_async_copy(
                x_hbm.at[metab[b].at[pl.ds(0, CHUNK)]], gbuf[b],
                sem_g).wait()
            if launch_ahead:
                launch(k + 2, (b + 2) % NBUF)
            scale_rows(gbuf[b], metab[b])
            pltpu.sync_copy(gbuf[b], h_sh.at[ridx[b]], add=True)
            if fetch_ahead:

                @pl.when(k + 3 < CHUNKS_PW)
                def _():
                    fetch_meta(k + 3, b)

        def round3(o, _):
            for b in range(NBUF):
                finish(o * NBUF + b, b, launch_ahead=True, fetch_ahead=True)
            return 0
        lax.fori_loop(0, (CHUNKS_PW - 2) // NBUF, round3, 0)
        finish(CHUNKS_PW - 2, (CHUNKS_PW - 2) % NBUF,
               launch_ahead=False, fetch_ahead=False)
        finish(CHUNKS_PW - 1, (CHUNKS_PW - 1) % NBUF,
               launch_ahead=False, fetch_ahead=False)

        plsc.subcore_barrier()

        for k in range(HROUNDS):
            cid = s + NS * k

            @pl.when(cid < N_HCHUNKS)
            def _():
                pltpu.sync_copy(h_sh.at[pl.ds(cid * HCHUNK, HCHUNK)],
                                out_hbm.at[c, pl.ds(cid * HCHUNK, HCHUNK)])

        @pl.when(s == NS - 1)
        def _():
            pltpu.sync_copy(h_sh.at[pl.ds(N_HCHUNKS * HCHUNK, HTAIL)],
                            out_hbm.at[c, pl.ds(N_HCHUNKS * HCHUNK, HTAIL)])

    return agg(x, meta)


def _tc_matmul_relu(h_partial, W):
    BLOCK = 1000

    def mm(h_ref, w_ref, o_ref):
        hp = h_ref[...]
        y = hp[0] + hp[1]
        o_ref[...] = jnp.maximum(
            jnp.dot(y, w_ref[...], preferred_element_type=jnp.float32), 0.0)

    return pl.pallas_call(
        mm,
        grid=(N_NODES // BLOCK,),
        in_specs=[
            pl.BlockSpec((NC, BLOCK, D), lambda i: (0, i, 0)),
            pl.BlockSpec((D, D), lambda i: (0, 0)),
        ],
        out_specs=pl.BlockSpec((BLOCK, D), lambda i: (i, 0)),
        out_shape=jax.ShapeDtypeStruct((N_NODES, D), jnp.float32),
    )(h_partial, W)


def _pack_meta(adj_indices, adj_values):
    rows = adj_indices[0]
    cols = adj_indices[1]
    vbits = lax.bitcast_convert_type(adj_values, jnp.int32)
    pad = PAD_EDGES - N_EDGES
    spread = jnp.arange(pad, dtype=jnp.int32) % N_NODES
    cols_p = jnp.concatenate([cols, spread]).reshape(-1, CHUNK)
    rows_p = jnp.concatenate([rows, spread]).reshape(-1, CHUNK)
    vbits_p = jnp.concatenate(
        [vbits, jnp.zeros((pad,), jnp.int32)]).reshape(-1, CHUNK)
    meta = jnp.stack([cols_p, rows_p, vbits_p], axis=1)
    return meta.reshape(-1)


def kernel(input, adj_indices, adj_values, W):
    meta = _pack_meta(adj_indices, adj_values)
    h_partial = _sc_aggregate(input, meta)
    return _tc_matmul_relu(h_partial, W)


# speedup vs baseline: 4.0027x; 1.2085x over previous
import functools

import jax
import jax.numpy as jnp
from jax import lax
from jax.experimental import pallas as pl
from jax.experimental.pallas import tpu as pltpu
from jax.experimental.pallas import tpu_sc as plsc

N_NODES = 10000
N_EDGES = 320000
D = 128

NC = 2
NS = 16
L = 16
NW = NC * NS

CHUNK = 128
CHUNKS_PW = 80
PAD_EDGES = NW * CHUNKS_PW * CHUNK
NBUF = 3
MW = 3 * CHUNK

HCHUNK = 128
N_HCHUNKS = N_NODES // HCHUNK
HROUNDS = (N_HCHUNKS + NS - 1) // NS
HTAIL = N_NODES - N_HCHUNKS * HCHUNK


def _sc_aggregate(x, meta):
    mesh = plsc.VectorSubcoreMesh(
        core_axis_name="c", subcore_axis_name="s",
        num_cores=NC, num_subcores=NS)

    @functools.partial(
        pl.kernel,
        out_type=jax.ShapeDtypeStruct((NC, N_NODES, D), jnp.float32),
        mesh=mesh,
        scratch_types=[
            pltpu.VMEM_SHARED((N_NODES, D), jnp.float32),
            pltpu.VMEM((CHUNK, D), jnp.float32),
            pltpu.VMEM((CHUNK, D), jnp.float32),
            pltpu.VMEM((CHUNK, D), jnp.float32),
            pltpu.VMEM((MW,), jnp.int32),
            pltpu.VMEM((MW,), jnp.int32),
            pltpu.VMEM((MW,), jnp.int32),
            pltpu.VMEM((CHUNK,), jnp.int32),
            pltpu.VMEM((CHUNK,), jnp.int32),
            pltpu.VMEM((CHUNK,), jnp.int32),
            pltpu.SemaphoreType.DMA,
            pltpu.SemaphoreType.DMA,
            pltpu.SemaphoreType.DMA,
        ],
    )
    def agg(x_hbm, meta_hbm, out_hbm,
            h_sh, gbuf0, gbuf1, gbuf2, metab0, metab1, metab2,
            ridx0, ridx1, ridx2, sem_g, sem_m, sem_sc):
        c = lax.axis_index("c")
        s = lax.axis_index("s")
        wid = c * NS + s
        gbuf = (gbuf0, gbuf1, gbuf2)
        metab = (metab0, metab1, metab2)
        ridx = (ridx0, ridx1, ridx2)

        def zero_row(r, _):
            for j in range(D // L):
                gbuf0[r, pl.ds(j * L, L)] = jnp.zeros((L,), jnp.float32)
            return 0
        lax.fori_loop(0, HCHUNK, zero_row, 0)
        for k in range(HROUNDS):
            cid = s + NS * k

            @pl.when(cid < N_HCHUNKS)
            def _():
                pltpu.sync_copy(gbuf0, h_sh.at[pl.ds(cid * HCHUNK, HCHUNK)])

        @pl.when(s == NS - 1)
        def _():
            pltpu.sync_copy(gbuf0.at[pl.ds(0, HTAIL)],
                            h_sh.at[pl.ds(N_HCHUNKS * HCHUNK, HTAIL)])
        plsc.subcore_barrier()

        def scale_rows(gb, mb):
            def body(g, _):
                v16 = lax.bitcast_convert_type(
                    mb[pl.ds(2 * CHUNK + g * L, L)], jnp.float32)
                for i in range(L):
                    r = g * L + i
                    vs = v16[i]
                    for j in range(D // L):
                        gb[r, pl.ds(j * L, L)] = gb[r, pl.ds(j * L, L)] * vs
                return 0
            lax.fori_loop(0, CHUNK // L, body, 0)

        def fetch_meta(k, b):
            base = (wid * CHUNKS_PW + k) * MW
            pltpu.async_copy(meta_hbm.at[pl.ds(base, MW)], metab[b], sem_m)

        def wait_scatter(b):
            pltpu.make_async_copy(
                gbuf[b], h_sh.at[ridx[b]], sem_sc).wait()

        def launch(k, b, wait_sc):
            pltpu.make_async_copy(
                meta_hbm.at[pl.ds(0, MW)], metab[b], sem_m).wait()
            if wait_sc:
                wait_scatter(b)
            for g in range(CHUNK // L):
                ridx[b][pl.ds(g * L, L)] = metab[b][
                    pl.ds(CHUNK + g * L, L)]
            pltpu.async_copy(
                x_hbm.at[metab[b].at[pl.ds(0, CHUNK)]], gbuf[b], sem_g)

        fetch_meta(0, 0)
        fetch_meta(1, 1)
        launch(0, 0, wait_sc=False)
        launch(1, 1, wait_sc=False)
        fetch_meta(2, 2)

        def finish(k, b, launch_ahead, la_wait_sc, fetch_ahead):
            pltpu.make_async_copy(
                x_hbm.at[metab[b].at[pl.ds(0, CHUNK)]], gbuf[b],
                sem_g).wait()
            if launch_ahead:
                launch(k + 2, (b + 2) % NBUF, wait_sc=la_wait_sc)
            scale_rows(gbuf[b], metab[b])
            pltpu.async_copy(gbuf[b], h_sh.at[ridx[b]], sem_sc, add=True)
            if fetch_ahead:

                @pl.when(k + 3 < CHUNKS_PW)
                def _():
                    fetch_meta(k + 3, b)

        finish(0, 0, True, False, True)
        finish(1, 1, True, True, True)
        finish(2, 2, True, True, True)

        def round3(o, _):
            for b in range(NBUF):
                finish(o * NBUF + b, b, True, True, True)
            return 0
        lax.fori_loop(1, (CHUNKS_PW - 2) // NBUF, round3, 0)
        finish(CHUNKS_PW - 2, (CHUNKS_PW - 2) % NBUF, False, False, False)
        finish(CHUNKS_PW - 1, (CHUNKS_PW - 1) % NBUF, False, False, False)
        for b in range(NBUF):
            wait_scatter(b)

        plsc.subcore_barrier()

        for k in range(HROUNDS):
            cid = s + NS * k

            @pl.when(cid < N_HCHUNKS)
            def _():
                pltpu.sync_copy(h_sh.at[pl.ds(cid * HCHUNK, HCHUNK)],
                                out_hbm.at[c, pl.ds(cid * HCHUNK, HCHUNK)])

        @pl.when(s == NS - 1)
        def _():
            pltpu.sync_copy(h_sh.at[pl.ds(N_HCHUNKS * HCHUNK, HTAIL)],
                            out_hbm.at[c, pl.ds(N_HCHUNKS * HCHUNK, HTAIL)])

    return agg(x, meta)


def _tc_matmul_relu(h_partial, W):
    BLOCK = 1000

    def mm(h_ref, w_ref, o_ref):
        hp = h_ref[...]
        y = hp[0] + hp[1]
        o_ref[...] = jnp.maximum(
            jnp.dot(y, w_ref[...], preferred_element_type=jnp.float32), 0.0)

    return pl.pallas_call(
        mm,
        grid=(N_NODES // BLOCK,),
        in_specs=[
            pl.BlockSpec((NC, BLOCK, D), lambda i: (0, i, 0)),
            pl.BlockSpec((D, D), lambda i: (0, 0)),
        ],
        out_specs=pl.BlockSpec((BLOCK, D), lambda i: (i, 0)),
        out_shape=jax.ShapeDtypeStruct((N_NODES, D), jnp.float32),
    )(h_partial, W)


def _pack_meta(adj_indices, adj_values):
    rows = adj_indices[0]
    cols = adj_indices[1]
    vbits = lax.bitcast_convert_type(adj_values, jnp.int32)
    pad = PAD_EDGES - N_EDGES
    spread = jnp.arange(pad, dtype=jnp.int32) % N_NODES
    cols_p = jnp.concatenate([cols, spread]).reshape(-1, CHUNK)
    rows_p = jnp.concatenate([rows, spread]).reshape(-1, CHUNK)
    vbits_p = jnp.concatenate(
        [vbits, jnp.zeros((pad,), jnp.int32)]).reshape(-1, CHUNK)
    meta = jnp.stack([cols_p, rows_p, vbits_p], axis=1)
    return meta.reshape(-1)


def kernel(input, adj_indices, adj_values, W):
    meta = _pack_meta(adj_indices, adj_values)
    h_partial = _sc_aggregate(input, meta)
    return _tc_matmul_relu(h_partial, W)


# speedup vs baseline: 4.1593x; 1.0391x over previous
import functools

import jax
import jax.numpy as jnp
from jax import lax
from jax.experimental import pallas as pl
from jax.experimental.pallas import tpu as pltpu
from jax.experimental.pallas import tpu_sc as plsc

N_NODES = 10000
N_EDGES = 320000
D = 128

NC = 2
NS = 16
L = 16
NW = NC * NS

CHUNK = 128
CHUNKS_PW = 80
PAD_EDGES = NW * CHUNKS_PW * CHUNK
NBUF = 3
MW = 3 * CHUNK

HCHUNK = 128
N_HCHUNKS = N_NODES // HCHUNK
HROUNDS = (N_HCHUNKS + NS - 1) // NS
HTAIL = N_NODES - N_HCHUNKS * HCHUNK


def _sc_aggregate(x, meta):
    mesh = plsc.VectorSubcoreMesh(
        core_axis_name="c", subcore_axis_name="s",
        num_cores=NC, num_subcores=NS)

    @functools.partial(
        pl.kernel,
        out_type=jax.ShapeDtypeStruct((NC, N_NODES, D), jnp.float32),
        mesh=mesh,
        scratch_types=[
            pltpu.VMEM_SHARED((N_NODES, D), jnp.float32),
            pltpu.VMEM((CHUNK, D), jnp.float32),
            pltpu.VMEM((CHUNK, D), jnp.float32),
            pltpu.VMEM((CHUNK, D), jnp.float32),
            pltpu.VMEM((MW,), jnp.int32),
            pltpu.VMEM((MW,), jnp.int32),
            pltpu.VMEM((MW,), jnp.int32),
            pltpu.VMEM((CHUNK,), jnp.int32),
            pltpu.VMEM((CHUNK,), jnp.int32),
            pltpu.VMEM((CHUNK,), jnp.int32),
            pltpu.SemaphoreType.DMA,
            pltpu.SemaphoreType.DMA,
            pltpu.SemaphoreType.DMA,
        ],
    )
    def agg(x_hbm, meta_hbm, out_hbm,
            h_sh, gbuf0, gbuf1, gbuf2, metab0, metab1, metab2,
            ridx0, ridx1, ridx2, sem_g, sem_m, sem_sc):
        c = lax.axis_index("c")
        s = lax.axis_index("s")
        wid = c * NS + s
        gbuf = (gbuf0, gbuf1, gbuf2)
        metab = (metab0, metab1, metab2)
        ridx = (ridx0, ridx1, ridx2)

        def zero_row(r, _):
            for j in range(D // L):
                gbuf0[r, pl.ds(j * L, L)] = jnp.zeros((L,), jnp.float32)
            return 0
        lax.fori_loop(0, HCHUNK, zero_row, 0)
        for k in range(HROUNDS):
            cid = s + NS * k

            @pl.when(cid < N_HCHUNKS)
            def _():
                pltpu.sync_copy(gbuf0, h_sh.at[pl.ds(cid * HCHUNK, HCHUNK)])

        @pl.when(s == NS - 1)
        def _():
            pltpu.sync_copy(gbuf0.at[pl.ds(0, HTAIL)],
                            h_sh.at[pl.ds(N_HCHUNKS * HCHUNK, HTAIL)])
        plsc.subcore_barrier()

        def scale_rows(gb, mb):
            def body(g, _):
                v16 = lax.bitcast_convert_type(
                    mb[pl.ds(2 * CHUNK + g * L, L)], jnp.float32)
                for i in range(L):
                    r = g * L + i
                    vs = v16[i]
                    for j in range(D // L):
                        gb[r, pl.ds(j * L, L)] = gb[r, pl.ds(j * L, L)] * vs
                return 0
            lax.fori_loop(0, CHUNK // L, body, 0)

        def fetch_meta(k, b):
            base = (wid * CHUNKS_PW + k) * MW
            pltpu.async_copy(meta_hbm.at[pl.ds(base, MW)], metab[b], sem_m)

        def wait_scatter(b):
            pltpu.make_async_copy(
                gbuf[b], h_sh.at[ridx[b]], sem_sc).wait()

        def launch(k, b, wait_sc):
            pltpu.make_async_copy(
                meta_hbm.at[pl.ds(0, MW)], metab[b], sem_m).wait()
            if wait_sc:
                wait_scatter(b)
            for g in range(CHUNK // L):
                ridx[b][pl.ds(g * L, L)] = metab[b][
                    pl.ds(CHUNK + g * L, L)]
            pltpu.async_copy(
                x_hbm.at[metab[b].at[pl.ds(0, CHUNK)]], gbuf[b], sem_g)

        fetch_meta(0, 0)
        fetch_meta(1, 1)
        launch(0, 0, wait_sc=False)
        launch(1, 1, wait_sc=False)
        fetch_meta(2, 2)

        def finish(k, b, launch_ahead, la_wait_sc, fetch_ahead):
            pltpu.make_async_copy(
                x_hbm.at[metab[b].at[pl.ds(0, CHUNK)]], gbuf[b],
                sem_g).wait()
            scale_rows(gbuf[b], metab[b])
            pltpu.async_copy(gbuf[b], h_sh.at[ridx[b]], sem_sc, add=True)
            if launch_ahead:
                launch(k + 2, (b + 2) % NBUF, wait_sc=la_wait_sc)
            if fetch_ahead:

                @pl.when(k + 3 < CHUNKS_PW)
                def _():
                    fetch_meta(k + 3, b)

        finish(0, 0, True, False, True)
        finish(1, 1, True, True, True)
        finish(2, 2, True, True, True)

        def round3(o, _):
            for b in range(NBUF):
                finish(o * NBUF + b, b, True, True, True)
            return 0
        lax.fori_loop(1, (CHUNKS_PW - 2) // NBUF, round3, 0)
        finish(CHUNKS_PW - 2, (CHUNKS_PW - 2) % NBUF, False, False, False)
        finish(CHUNKS_PW - 1, (CHUNKS_PW - 1) % NBUF, False, False, False)
        for b in range(NBUF):
            wait_scatter(b)

        plsc.subcore_barrier()

        for k in range(HROUNDS):
            cid = s + NS * k

            @pl.when(cid < N_HCHUNKS)
            def _():
                pltpu.sync_copy(h_sh.at[pl.ds(cid * HCHUNK, HCHUNK)],
                                out_hbm.at[c, pl.ds(cid * HCHUNK, HCHUNK)])

        @pl.when(s == NS - 1)
        def _():
            pltpu.sync_copy(h_sh.at[pl.ds(N_HCHUNKS * HCHUNK, HTAIL)],
                            out_hbm.at[c, pl.ds(N_HCHUNKS * HCHUNK, HTAIL)])

    return agg(x, meta)


def _tc_matmul_relu(h_partial, W):
    BLOCK = 1000

    def mm(h_ref, w_ref, o_ref):
        hp = h_ref[...]
        y = hp[0] + hp[1]
        o_ref[...] = jnp.maximum(
            jnp.dot(y, w_ref[...], preferred_element_type=jnp.float32), 0.0)

    return pl.pallas_call(
        mm,
        grid=(N_NODES // BLOCK,),
        in_specs=[
            pl.BlockSpec((NC, BLOCK, D), lambda i: (0, i, 0)),
            pl.BlockSpec((D, D), lambda i: (0, 0)),
        ],
        out_specs=pl.BlockSpec((BLOCK, D), lambda i: (i, 0)),
        out_shape=jax.ShapeDtypeStruct((N_NODES, D), jnp.float32),
    )(h_partial, W)


def _pack_meta(adj_indices, adj_values):
    rows = adj_indices[0]
    cols = adj_indices[1]
    vbits = lax.bitcast_convert_type(adj_values, jnp.int32)
    pad = PAD_EDGES - N_EDGES
    spread = jnp.arange(pad, dtype=jnp.int32) % N_NODES
    cols_p = jnp.concatenate([cols, spread]).reshape(-1, CHUNK)
    rows_p = jnp.concatenate([rows, spread]).reshape(-1, CHUNK)
    vbits_p = jnp.concatenate(
        [vbits, jnp.zeros((pad,), jnp.int32)]).reshape(-1, CHUNK)
    meta = jnp.stack([cols_p, rows_p, vbits_p], axis=1)
    return meta.reshape(-1)


def kernel(input, adj_indices, adj_values, W):
    meta = _pack_meta(adj_indices, adj_values)
    h_partial = _sc_aggregate(input, meta)
    return _tc_matmul_relu(h_partial, W)
